# baseline (speedup 1.0000x reference)
import functools

import jax
import jax.numpy as jnp
from jax import lax
from jax.experimental import pallas as pl
from jax.experimental.pallas import tpu as pltpu
from jax.experimental.pallas import tpu_sc as plsc

B = 16384
D = 64
NT = 6

V_PIDS, V_CATE, V_CUST, V_BRAND, V_CAMP, V_PRICE = 2, 806, 935, 846, 411, 11


def _even(v):
    return v + (v % 2)


A_ROW_OFF = (0, V_PIDS, V_PIDS + V_CATE, V_PIDS + V_CATE + _even(V_CAMP))
TBL_ROWS128 = (V_PIDS + V_CATE + _even(V_CAMP) + _even(V_PRICE)) * D // 128

N_CB = 2048
N_A = 1024
CB_CHUNK = 128
A_CHUNK = 32

_info = plsc.get_sparse_core_info()
_NC = _info.num_cores
_NS = _info.num_subcores

_mesh = plsc.VectorSubcoreMesh(core_axis_name="c", subcore_axis_name="s")


@functools.partial(
    pl.kernel,
    mesh=_mesh,
    compiler_params=pltpu.CompilerParams(use_tc_tiling_on_sc=False),
    out_type=jax.ShapeDtypeStruct((B, NT * D), jnp.float32),
    scratch_types=[
        pltpu.VMEM((TBL_ROWS128, 128), jnp.float32),
        pltpu.VMEM((32, 128), jnp.int32),
        pltpu.VMEM((2, CB_CHUNK, D), jnp.float32),
        pltpu.VMEM((2, A_CHUNK, 2 * D), jnp.float32),
        pltpu.VMEM((2, A_CHUNK, 2 * D), jnp.float32),
        pltpu.SemaphoreType.DMA,
    ],
)
def _emb_kernel(pids_h, cate_h, cust_h, brand_h, camp_h, price_h,
                wpids_h, wcate_h, wcust_h, wbrand_h, wcamp_h, wprice_h,
                out_h, tbl, idx4, obuf_cb, obuf01, obuf45, wsem):
    wid = lax.axis_index("s") * _NC + lax.axis_index("c")

    def gather_row(buf, p, i, tblrow, coloff):
        row = tblrow >> 1
        col = (tblrow & 1) * D
        for k in range(D // 16):
            buf[p, i, pl.ds(coloff + k * 16, 16)] = tbl[row,
                                                        pl.ds(col + k * 16, 16)]

    def gather16(buf, p, irow0, vidx, coloff, rowoff):
        for j in range(16):
            gather_row(buf, p, irow0 + j, vidx[j] + rowoff, coloff)

    def load_vidx(r):
        return idx4[r >> 7, pl.ds(lax.rem(r, 128), 16)]

    def cb_group(idx2_h, wtbl_h, rows128, local, tcol):
        pltpu.sync_copy(wtbl_h, tbl.at[pl.ds(0, rows128)])
        pltpu.sync_copy(idx2_h.at[pl.ds(local * (N_CB // 128), N_CB // 128)],
                        idx4.at[pl.ds(0, N_CB // 128)])
        nch = N_CB // CB_CHUNK

        def gather_chunk(c, slot):
            def body(g, carry):
                vidx = load_vidx(c * CB_CHUNK + g * 16)
                gather16(obuf_cb, slot, g * 16, vidx, 0, 0)
                return carry

            lax.fori_loop(0, CB_CHUNK // 16, body, 0)

        def fire_write(c, slot):
            b0 = local * N_CB + c * CB_CHUNK
            return pltpu.async_copy(
                obuf_cb.at[slot],
                out_h.at[pl.ds(b0, CB_CHUNK), pl.ds(tcol, D)], wsem)

        def drain(slot):
            pltpu.make_async_copy(
                obuf_cb.at[slot],
                out_h.at[pl.ds(local * N_CB, CB_CHUNK), pl.ds(tcol, D)],
                wsem).wait()

        for b in range(2):
            gather_chunk(b, b)
            fire_write(b, b)

        def outer(i, carry):
            for b in range(2):
                c = 2 * i + 2 + b
                drain(b)
                gather_chunk(c, b)
                fire_write(c, b)
            return carry

        lax.fori_loop(0, (nch - 2) // 2, outer, 0)
        for b in range(2):
            drain(b)

    def a_group(local):
        srcs = (wpids_h, wcate_h, wcamp_h, wprice_h)
        for s in range(4):
            r0 = A_ROW_OFF[s] * D // 128
            pltpu.sync_copy(srcs[s], tbl.at[pl.ds(r0, srcs[s].shape[0])])
        idxs = (pids_h, cate_h, camp_h, price_h)
        nr = N_A // 128
        for s in range(4):
            pltpu.sync_copy(idxs[s].at[pl.ds(local * nr, nr)],
                            idx4.at[pl.ds(s * nr, nr)])
        nch = N_A // A_CHUNK

        def gather_chunk(c, slot):
            def body(g, carry):
                r0 = c * A_CHUNK + g * 16
                gather16(obuf01, slot, g * 16, load_vidx(0 * N_A + r0), 0,
                         A_ROW_OFF[0])
                gather16(obuf01, slot, g * 16, load_vidx(1 * N_A + r0), D,
                         A_ROW_OFF[1])
                gather16(obuf45, slot, g * 16, load_vidx(2 * N_A + r0), 0,
                         A_ROW_OFF[2])
                gather16(obuf45, slot, g * 16, load_vidx(3 * N_A + r0), D,
                         A_ROW_OFF[3])
                return carry

            lax.fori_loop(0, A_CHUNK // 16, body, 0)

        def fire_writes(c, slot):
            b0 = local * N_A + c * A_CHUNK
            pltpu.async_copy(
                obuf01.at[slot],
                out_h.at[pl.ds(b0, A_CHUNK), pl.ds(0, 2 * D)], wsem)
            pltpu.async_copy(
                obuf45.at[slot],
                out_h.at[pl.ds(b0, A_CHUNK), pl.ds(4 * D, 2 * D)], wsem)

        def drain(slot):
            for buf in (obuf01, obuf45):
                pltpu.make_async_copy(
                    buf.at[slot],
                    out_h.at[pl.ds(local * N_A, A_CHUNK), pl.ds(0, 2 * D)],
                    wsem).wait()

        for b in range(2):
            gather_chunk(b, b)
            fire_writes(b, b)

        def outer(i, carry):
            for b in range(2):
                c = 2 * i + 2 + b
                drain(b)
                gather_chunk(c, b)
                fire_writes(c, b)
            return carry

        lax.fori_loop(0, (nch - 2) // 2, outer, 0)
        for b in range(2):
            drain(b)

    @pl.when(wid < 8)
    def _():
        cb_group(cust_h, wcust_h, _even(V_CUST) * D // 128, wid, 2 * D)

    @pl.when((wid >= 8) & (wid < 16))
    def _():
        cb_group(brand_h, wbrand_h, V_BRAND * D // 128, wid - 8, 3 * D)

    @pl.when(wid >= 16)
    def _():
        a_group(wid - 16)


def _w128(w):
    v = w.shape[0]
    if v % 2:
        w = jnp.concatenate([w, jnp.zeros((1, D), w.dtype)], axis=0)
    return w.reshape(-1, 128)


def kernel(cate, customer, brand, campaign, price, pids, id, W_cate,
           W_customer, W_brand, W_campaign, W_price, W_pids, W_id):
    shp = (B // 128, 128)
    out = _emb_kernel(
        pids.reshape(shp), cate.reshape(shp), customer.reshape(shp),
        brand.reshape(shp), campaign.reshape(shp), price.reshape(shp),
        _w128(W_pids), _w128(W_cate), _w128(W_customer),
        _w128(W_brand), _w128(W_campaign), _w128(W_price))
    return out.reshape(B, NT, D)


# speedup vs baseline: 1.0793x; 1.0793x over previous
import functools

import jax
import jax.numpy as jnp
from jax import lax
from jax.experimental import pallas as pl
from jax.experimental.pallas import tpu as pltpu
from jax.experimental.pallas import tpu_sc as plsc

B = 16384
D = 64
NT = 6

V_PIDS, V_CATE, V_CUST, V_BRAND, V_CAMP, V_PRICE = 2, 806, 935, 846, 411, 11


def _even(v):
    return v + (v % 2)


A_ROW_OFF = (0, V_PIDS, V_PIDS + V_CATE, V_PIDS + V_CATE + _even(V_CAMP))
TBL_ROWS128 = (V_PIDS + V_CATE + _even(V_CAMP) + _even(V_PRICE)) * D // 128

N_CB = 2048
N_A = 1024
CB_CHUNK = 128
A_CHUNK = 32

_info = plsc.get_sparse_core_info()
_NC = _info.num_cores
_NS = _info.num_subcores

_mesh = plsc.VectorSubcoreMesh(core_axis_name="c", subcore_axis_name="s")


@functools.partial(
    pl.kernel,
    mesh=_mesh,
    compiler_params=pltpu.CompilerParams(use_tc_tiling_on_sc=False),
    out_type=[jax.ShapeDtypeStruct((B, 128), jnp.float32)] * 3,
    scratch_types=[
        pltpu.VMEM((TBL_ROWS128, 128), jnp.float32),
        pltpu.VMEM((32, 128), jnp.int32),
        pltpu.VMEM((2, CB_CHUNK, D), jnp.float32),
        pltpu.VMEM((2, A_CHUNK, 2 * D), jnp.float32),
        pltpu.VMEM((2, A_CHUNK, 2 * D), jnp.float32),
        pltpu.SemaphoreType.DMA,
    ],
)
def _emb_kernel(pids_h, cate_h, cust_h, brand_h, camp_h, price_h,
                wpids_h, wcate_h, wcust_h, wbrand_h, wcamp_h, wprice_h,
                out01_h, out23_h, out45_h, tbl, idx4, obuf_cb, obuf01, obuf45,
                wsem):
    wid = lax.axis_index("s") * _NC + lax.axis_index("c")

    def gather_row(buf, p, i, tblrow, coloff):
        row = tblrow >> 1
        col = (tblrow & 1) * D
        for k in range(D // 16):
            buf[p, i, pl.ds(coloff + k * 16, 16)] = tbl[row,
                                                        pl.ds(col + k * 16, 16)]

    def gather16(buf, p, irow0, vidx, coloff, rowoff):
        for j in range(16):
            gather_row(buf, p, irow0 + j, vidx[j] + rowoff, coloff)

    def load_vidx(r):
        return idx4[r >> 7, pl.ds(lax.rem(r, 128), 16)]

    def cb_group(idx2_h, wtbl_h, rows128, local, tcol):
        pltpu.sync_copy(wtbl_h, tbl.at[pl.ds(0, rows128)])
        pltpu.sync_copy(idx2_h.at[pl.ds(local * (N_CB // 128), N_CB // 128)],
                        idx4.at[pl.ds(0, N_CB // 128)])
        nch = N_CB // CB_CHUNK

        def gather_chunk(c, slot):
            def body(g, carry):
                vidx = load_vidx(c * CB_CHUNK + g * 16)
                gather16(obuf_cb, slot, g * 16, vidx, 0, 0)
                return carry

            lax.fori_loop(0, CB_CHUNK // 16, body, 0)

        def fire_write(c, slot):
            b0 = local * N_CB + c * CB_CHUNK
            return pltpu.async_copy(
                obuf_cb.at[slot],
                out23_h.at[pl.ds(b0, CB_CHUNK), pl.ds(tcol, D)], wsem)

        def drain(slot):
            pltpu.make_async_copy(
                obuf_cb.at[slot],
                out23_h.at[pl.ds(local * N_CB, CB_CHUNK), pl.ds(tcol, D)],
                wsem).wait()

        for b in range(2):
            gather_chunk(b, b)
            fire_write(b, b)

        def outer(i, carry):
            for b in range(2):
                c = 2 * i + 2 + b
                drain(b)
                gather_chunk(c, b)
                fire_write(c, b)
            return carry

        lax.fori_loop(0, (nch - 2) // 2, outer, 0)
        for b in range(2):
            drain(b)

    def a_group(local):
        srcs = (wpids_h, wcate_h, wcamp_h, wprice_h)
        for s in range(4):
            r0 = A_ROW_OFF[s] * D // 128
            pltpu.sync_copy(srcs[s], tbl.at[pl.ds(r0, srcs[s].shape[0])])
        idxs = (pids_h, cate_h, camp_h, price_h)
        nr = N_A // 128
        for s in range(4):
            pltpu.sync_copy(idxs[s].at[pl.ds(local * nr, nr)],
                            idx4.at[pl.ds(s * nr, nr)])
        nch = N_A // A_CHUNK

        def gather_chunk(c, slot):
            def body(g, carry):
                r0 = c * A_CHUNK + g * 16
                gather16(obuf01, slot, g * 16, load_vidx(0 * N_A + r0), 0,
                         A_ROW_OFF[0])
                gather16(obuf01, slot, g * 16, load_vidx(1 * N_A + r0), D,
                         A_ROW_OFF[1])
                gather16(obuf45, slot, g * 16, load_vidx(2 * N_A + r0), 0,
                         A_ROW_OFF[2])
                gather16(obuf45, slot, g * 16, load_vidx(3 * N_A + r0), D,
                         A_ROW_OFF[3])
                return carry

            lax.fori_loop(0, A_CHUNK // 16, body, 0)

        def fire_writes(c, slot):
            b0 = local * N_A + c * A_CHUNK
            pltpu.async_copy(obuf01.at[slot],
                             out01_h.at[pl.ds(b0, A_CHUNK)], wsem)
            pltpu.async_copy(obuf45.at[slot],
                             out45_h.at[pl.ds(b0, A_CHUNK)], wsem)

        def drain(slot):
            for buf, oh in ((obuf01, out01_h), (obuf45, out45_h)):
                pltpu.make_async_copy(
                    buf.at[slot], oh.at[pl.ds(local * N_A, A_CHUNK)],
                    wsem).wait()

        for b in range(2):
            gather_chunk(b, b)
            fire_writes(b, b)

        def outer(i, carry):
            for b in range(2):
                c = 2 * i + 2 + b
                drain(b)
                gather_chunk(c, b)
                fire_writes(c, b)
            return carry

        lax.fori_loop(0, (nch - 2) // 2, outer, 0)
        for b in range(2):
            drain(b)

    @pl.when(wid < 8)
    def _():
        cb_group(cust_h, wcust_h, _even(V_CUST) * D // 128, wid, 0)

    @pl.when((wid >= 8) & (wid < 16))
    def _():
        cb_group(brand_h, wbrand_h, V_BRAND * D // 128, wid - 8, D)

    @pl.when(wid >= 16)
    def _():
        a_group(wid - 16)


def _w128(w):
    v = w.shape[0]
    if v % 2:
        w = jnp.concatenate([w, jnp.zeros((1, D), w.dtype)], axis=0)
    return w.reshape(-1, 128)


def kernel(cate, customer, brand, campaign, price, pids, id, W_cate,
           W_customer, W_brand, W_campaign, W_price, W_pids, W_id):
    shp = (B // 128, 128)
    out01, out23, out45 = _emb_kernel(
        pids.reshape(shp), cate.reshape(shp), customer.reshape(shp),
        brand.reshape(shp), campaign.reshape(shp), price.reshape(shp),
        _w128(W_pids), _w128(W_cate), _w128(W_customer),
        _w128(W_brand), _w128(W_campaign), _w128(W_price))
    return jnp.stack([out01, out23, out45], axis=1).reshape(B, NT, D)


# speedup vs baseline: 1.2298x; 1.1394x over previous
import functools

import jax
import jax.numpy as jnp
from jax import lax
from jax.experimental import pallas as pl
from jax.experimental.pallas import tpu as pltpu
from jax.experimental.pallas import tpu_sc as plsc

B = 16384
D = 64
NT = 6

V_PIDS, V_CATE, V_CUST, V_BRAND, V_CAMP, V_PRICE = 2, 806, 935, 846, 411, 11


def _pad16(v):
    return (v + 15) // 16 * 16


G_ROWS = ((_pad16(V_PIDS), _pad16(V_CATE)),
          (_pad16(V_CUST), _pad16(V_BRAND)),
          (_pad16(V_CAMP), _pad16(V_PRICE)))
TBL_ROWS128 = max(a + b for a, b in G_ROWS) * D // 128

G_TILES = (11, 11, 10)
G_FIRST = (0, 11, 22)
NPAIRS = B // 32
CHUNK = 16
IDXROWS = 24

_info = plsc.get_sparse_core_info()
_NC = _info.num_cores
_NS = _info.num_subcores

_mesh = plsc.VectorSubcoreMesh(core_axis_name="c", subcore_axis_name="s")


@functools.partial(
    pl.kernel,
    mesh=_mesh,
    compiler_params=pltpu.CompilerParams(use_tc_tiling_on_sc=True),
    out_type=[jax.ShapeDtypeStruct((B, 128), jnp.float32)] * 3,
    scratch_types=[
        pltpu.VMEM((TBL_ROWS128, 128), jnp.float32),
        pltpu.VMEM((2 * IDXROWS, 128), jnp.int32),
        pltpu.VMEM((2, CHUNK, 128), jnp.float32),
        pltpu.SemaphoreType.DMA,
    ],
)
def _emb_kernel(pids_h, cate_h, cust_h, brand_h, camp_h, price_h,
                wpids_h, wcate_h, wcust_h, wbrand_h, wcamp_h, wprice_h,
                out01_h, out23_h, out45_h, tbl, idx2, obuf, wsem):
    wid = lax.axis_index("s") * _NC + lax.axis_index("c")

    def gather_row(p, i, tblrow, coloff):
        row = tblrow >> 1
        col = (tblrow & 1) * D
        for k in range(D // 16):
            obuf[p, i, pl.ds(coloff + k * 16, 16)] = tbl[row,
                                                         pl.ds(col + k * 16,
                                                               16)]

    def group(gi, idxe_h, idxo_h, we_h, wo_h, out_h):
        local = wid - G_FIRST[gi]
        ntiles = G_TILES[gi]
        basep, remp = NPAIRS // ntiles, NPAIRS % ntiles
        lo = (local * basep + lax.min(local, remp)) * 32
        npair = basep + jnp.where(local < remp, 1, 0)

        rows0 = we_h.shape[0]
        pltpu.sync_copy(we_h, tbl.at[pl.ds(0, rows0)])
        pltpu.sync_copy(wo_h, tbl.at[pl.ds(rows0, wo_h.shape[0])])
        rowoff_o = rows0 * 128 // D

        r8 = lax.min((lo // 1024) * 8, (B // 128) - IDXROWS)
        ofs = lo - r8 * 128
        pltpu.sync_copy(idxe_h.at[pl.ds(r8, IDXROWS)],
                        idx2.at[pl.ds(0, IDXROWS)])
        pltpu.sync_copy(idxo_h.at[pl.ds(r8, IDXROWS)],
                        idx2.at[pl.ds(IDXROWS, IDXROWS)])

        def vidx(tab, r):
            f = ofs + r
            return idx2[tab * IDXROWS + (f >> 7), pl.ds(lax.rem(f, 128), 16)]

        def gather_chunk(c, slot):
            r0 = c * CHUNK
            ve = vidx(0, r0)
            vo = vidx(1, r0)
            for j in range(CHUNK):
                gather_row(slot, j, ve[j], 0)
                gather_row(slot, j, vo[j] + rowoff_o, D)

        def fire_write(c, slot):
            pltpu.async_copy(obuf.at[slot],
                             out_h.at[pl.ds(lo + c * CHUNK, CHUNK)], wsem)

        def drain(slot):
            pltpu.make_async_copy(obuf.at[slot],
                                  out_h.at[pl.ds(lo, CHUNK)], wsem).wait()

        for b in range(2):
            gather_chunk(b, b)
            fire_write(b, b)

        def outer(i, carry):
            for b in range(2):
                c = 2 * i + 2 + b
                drain(b)
                gather_chunk(c, b)
                fire_write(c, b)
            return carry

        lax.fori_loop(0, npair - 1, outer, 0)
        for b in range(2):
            drain(b)

    @pl.when(wid < 11)
    def _():
        group(0, pids_h, cate_h, wpids_h, wcate_h, out01_h)

    @pl.when((wid >= 11) & (wid < 22))
    def _():
        group(1, cust_h, brand_h, wcust_h, wbrand_h, out23_h)

    @pl.when(wid >= 22)
    def _():
        group(2, camp_h, price_h, wcamp_h, wprice_h, out45_h)


def _w128(w, rows):
    v = w.shape[0]
    if rows != v:
        w = jnp.concatenate([w, jnp.zeros((rows - v, D), w.dtype)], axis=0)
    return w.reshape(-1, 128)


def kernel(cate, customer, brand, campaign, price, pids, id, W_cate,
           W_customer, W_brand, W_campaign, W_price, W_pids, W_id):
    shp = (B // 128, 128)
    out01, out23, out45 = _emb_kernel(
        pids.reshape(shp), cate.reshape(shp), customer.reshape(shp),
        brand.reshape(shp), campaign.reshape(shp), price.reshape(shp),
        _w128(W_pids, G_ROWS[0][0]), _w128(W_cate, G_ROWS[0][1]),
        _w128(W_customer, G_ROWS[1][0]), _w128(W_brand, G_ROWS[1][1]),
        _w128(W_campaign, G_ROWS[2][0]), _w128(W_price, G_ROWS[2][1]))
    return jnp.stack([out01, out23, out45], axis=1).reshape(B, NT, D)
